# baseline (speedup 1.0000x reference)
import jax
import jax.numpy as jnp
from jax import lax
from jax.experimental import pallas as pl
from jax.experimental.pallas import tpu as pltpu
from jax.experimental.pallas import tpu_sc as plsc

B = 16384
D = 32
OUT_D = 3 * D + 1
V_ID = 1000000

NC = 2
NS = 16
NW = NC * NS
BPW = B // NW
CHUNK = 128
NCH = BPW // CHUNK
L = 16

BLKL = 128
NBLK = 7813
CPW = NBLK // NW
XTRA = NBLK % NW
RING = 8


def _conv_body(tab_t, id_f, slab, sem_in, sem_out):
    c = lax.axis_index("c")
    s = lax.axis_index("s")
    wid = s * NC + c
    start = wid * CPW + jnp.minimum(wid, XTRA)
    cnt = CPW + (wid < XTRA).astype(jnp.int32)

    PD = RING - 2
    for p in range(PD):
        @pl.when(p < cnt)
        def _():
            pltpu.async_copy(
                tab_t.at[:, pl.ds((start + p) * BLKL, BLKL)], slab.at[p],
                sem_in)

    def step(i, carry):
        b = lax.rem(i, RING)
        nb = lax.rem(i + PD, RING)
        blk = start + i
        pltpu.make_async_copy(
            tab_t.at[:, pl.ds(0, BLKL)], slab.at[b], sem_in).wait()

        @pl.when(i >= 2)
        def _():
            pltpu.make_async_copy(slab.at[0], id_f.at[0], sem_out).wait()

        @pl.when(i + PD < cnt)
        def _():
            pltpu.async_copy(
                tab_t.at[:, pl.ds((blk + PD) * BLKL, BLKL)],
                slab.at[nb], sem_in)

        pltpu.async_copy(slab.at[b], id_f.at[blk], sem_out)
        return carry

    lax.fori_loop(0, cnt, step, None)
    for _ in range(2):
        pltpu.make_async_copy(slab.at[0], id_f.at[0], sem_out).wait()


@jax.jit
def _conv(tab_t):
    mesh = plsc.VectorSubcoreMesh(core_axis_name="c", subcore_axis_name="s")
    return pl.kernel(
        _conv_body,
        out_type=jax.ShapeDtypeStruct((NBLK, D, BLKL), jnp.float32),
        mesh=mesh,
        compiler_params=pltpu.CompilerParams(
            use_tc_tiling_on_sc=True, needs_layout_passes=False),
        scratch_types=[
            pltpu.VMEM((RING, D, BLKL), jnp.float32),
            pltpu.SemaphoreType.DMA,
            pltpu.SemaphoreType.DMA,
        ],
    )(tab_t)


def _gather_body(cid_h, memi_h, zipi_h, age_h, scale_h,
                 id_flat, mem_tab, zip_tab, out_t,
                 cid_v, memi_v, zipi_v, age_v, scale_v, memtab_v,
                 idxbuf, rows_zip, outbuf, sem_id, sem_row):
    c = lax.axis_index("c")
    s = lax.axis_index("s")
    wid = s * NC + c
    cbase = wid * NCH
    base = wid * BPW

    pltpu.sync_copy(cid_h.at[pl.ds(cbase, NCH)], cid_v)
    stage = [
        pltpu.async_copy(memi_h.at[pl.ds(cbase, NCH)], memi_v, sem_row),
        pltpu.async_copy(zipi_h.at[pl.ds(cbase, NCH)], zipi_v, sem_row),
        pltpu.async_copy(age_h.at[pl.ds(cbase, NCH)], age_v, sem_row),
        pltpu.async_copy(scale_h, scale_v, sem_row),
        pltpu.async_copy(mem_tab, memtab_v, sem_row),
    ]

    for j in range(NCH):
        for k in range(CHUNK // L):
            v = cid_v[j, pl.ds(k * L, L)]
            eb = ((v >> 7) << 12) + (v & 127)
            idxbuf[j, 0, pl.ds(k * L, L)] = eb
        pltpu.async_copy(
            id_flat.at[idxbuf.at[j, 0]],
            outbuf.at[0, pl.ds(j * CHUNK, CHUNK)], sem_id)

        def fire_d(d, carry):
            def addr(kk, carry2):
                e = idxbuf[j, 0, pl.ds(kk * L, L)]
                idxbuf[j, d, pl.ds(kk * L, L)] = e + d * BLKL
                return carry2
            lax.fori_loop(0, CHUNK // L, addr, None, unroll=4)
            pltpu.async_copy(
                id_flat.at[idxbuf.at[j, d]],
                outbuf.at[d, pl.ds(j * CHUNK, CHUNK)], sem_id)
            return carry

        lax.fori_loop(1, D, fire_d, None)

    for cp in stage:
        cp.wait()

    row_copies = []
    for j in range(NCH):
        row_copies.append(pltpu.async_copy(
            zip_tab.at[zipi_v.at[j]], rows_zip.at[pl.ds(j * CHUNK, CHUNK)],
            sem_row))

    mean = scale_v[pl.ds(0, L)]
    inv = scale_v[pl.ds(L, L)]
    for j in range(NCH):
        for k in range(CHUNK // L):
            a = age_v[j, pl.ds(k * L, L)]
            outbuf[3 * D, pl.ds(j * CHUNK + k * L, L)] = (a - mean) * inv

    for j in range(NCH):
        for k in range(CHUNK // L):
            m16 = memi_v[j, pl.ds(k * L, L)]
            col = j * CHUNK + k * L
            for d in range(D):
                vals = plsc.load_gather(
                    memtab_v, [m16, jnp.full((L,), d, jnp.int32)])
                outbuf[D + d, pl.ds(col, L)] = vals

    for cp in row_copies:
        cp.wait()

    lane = lax.iota(jnp.int32, L)

    def trans(r0, carry):
        ridx = lane + r0 * L
        for d in range(D):
            zvals = plsc.load_gather(
                rows_zip, [ridx, jnp.full((L,), d, jnp.int32)])
            outbuf[2 * D + d, pl.ds(r0 * L, L)] = zvals
        return carry

    lax.fori_loop(0, BPW // L, trans, None)

    def drain(t, carry):
        pltpu.make_async_copy(
            id_flat.at[pl.ds(0, CHUNK)],
            outbuf.at[0, pl.ds(0, CHUNK)], sem_id).wait()
        return carry

    lax.fori_loop(0, NCH * D, drain, None, unroll=4)

    pltpu.sync_copy(outbuf, out_t.at[:, pl.ds(base, BPW)])


@jax.jit
def _impl(cid2, memi2, zipi2, age2, scale, id_flat, membership_table,
          zip_table):
    mesh = plsc.VectorSubcoreMesh(core_axis_name="c", subcore_axis_name="s")
    return pl.kernel(
        _gather_body,
        out_type=jax.ShapeDtypeStruct((OUT_D, B), jnp.float32),
        mesh=mesh,
        compiler_params=pltpu.CompilerParams(
            use_tc_tiling_on_sc=False, needs_layout_passes=False),
        scratch_types=[
            pltpu.VMEM((NCH, CHUNK), jnp.int32),
            pltpu.VMEM((NCH, CHUNK), jnp.int32),
            pltpu.VMEM((NCH, CHUNK), jnp.int32),
            pltpu.VMEM((NCH, CHUNK), jnp.float32),
            pltpu.VMEM((2 * L,), jnp.float32),
            pltpu.VMEM((8, D), jnp.float32),
            pltpu.VMEM((NCH, D, CHUNK), jnp.int32),
            pltpu.VMEM((BPW, D), jnp.float32),
            pltpu.VMEM((OUT_D, BPW), jnp.float32),
            pltpu.SemaphoreType.DMA,
            pltpu.SemaphoreType.DMA,
        ],
    )(cid2, memi2, zipi2, age2, scale, id_flat, membership_table, zip_table)


def kernel(customer_id, club_member_status, postal_code, age,
           id_table, membership_table, zip_table, age_mean, age_var):
    inv_std = lax.rsqrt(age_var.astype(jnp.float32))
    scale = jnp.concatenate([
        jnp.full((L,), age_mean, jnp.float32),
        jnp.full((L,), inv_std, jnp.float32),
    ])
    cid2 = customer_id.reshape(NW * NCH, CHUNK)
    memi2 = club_member_status.reshape(NW * NCH, CHUNK)
    zipi2 = postal_code.reshape(NW * NCH, CHUNK)
    age2 = age.reshape(NW * NCH, CHUNK)
    id_flat = _conv(id_table.T).reshape(-1)
    out_t = _impl(cid2, memi2, zipi2, age2, scale, id_flat,
                  membership_table, zip_table)
    return out_t.T
